# baseline (speedup 1.0000x reference)
import functools

import jax
import jax.numpy as jnp
from jax import lax
from jax.experimental import pallas as pl
from jax.experimental.pallas import tpu as pltpu
from jax.experimental.pallas import tpu_sc as plsc

DIM = 2048
NUM_EXPERTS = 64
EXPERTS_PER_TOKEN = 8
NUM_GROUPS = 8
EXPERTS_PER_GROUP = NUM_EXPERTS // NUM_GROUPS
TOPK_GROUPS = 4
SCALE = 2.5
N_TOKENS = 8192
BLK = 256
NBLK = N_TOKENS // BLK
TOTAL_SEL = N_TOKENS * EXPERTS_PER_TOKEN

NEG_INF = float("-inf")


def _partner(v, k):
    left = pltpu.roll(v, NUM_EXPERTS - k, axis=1)
    right = pltpu.roll(v, k, axis=1)
    lane = lax.broadcasted_iota(jnp.int32, v.shape, 1)
    return jnp.where(lane & k == 0, left, right)


def _group_reduce(v, op):
    for k in (1, 2, 4):
        v = op(v, _partner(v, k))
    return v


def _routing_body(x_ref, w_ref, b_ref,
                  code_out, wgt_out, counts_out, start_out,
                  carry_ref):
    blk = pl.program_id(0)
    logits = jnp.dot(x_ref[...], w_ref[...], preferred_element_type=jnp.float32)
    scores = jax.nn.sigmoid(logits) + b_ref[...]

    lane = lax.broadcasted_iota(jnp.int32, (BLK, NUM_EXPERTS), 1)
    lane_f = lane.astype(jnp.float32)

    ei = lax.broadcasted_iota(jnp.int32, (NUM_EXPERTS, NUM_EXPERTS), 0)
    ej = lax.broadcasted_iota(jnp.int32, (NUM_EXPERTS, NUM_EXPERTS), 1)
    utri = (ei < ej).astype(jnp.float32)
    gtri = ((ei < ej) & (ei // EXPERTS_PER_GROUP == ej // EXPERTS_PER_GROUP)
            ).astype(jnp.float32)
    ones64 = jnp.ones((NUM_EXPERTS, NUM_EXPERTS), jnp.float32)

    m1 = _group_reduce(scores, jnp.maximum)
    eq1 = scores == m1
    pfx1 = jnp.dot(jnp.where(eq1, 1.0, 0.0), gtri,
                   preferred_element_type=jnp.float32)
    foc1 = jnp.logical_and(eq1, pfx1 == 0.0)
    m2 = _group_reduce(jnp.where(foc1, NEG_INF, scores), jnp.maximum)
    gs = m1 + m2

    grank = jnp.zeros((BLK, NUM_EXPERTS), jnp.float32)
    for j in range(1, NUM_GROUPS):
        r = pltpu.roll(gs, EXPERTS_PER_GROUP * j, axis=1)
        beats = jnp.logical_or(r > gs,
                               jnp.logical_and(r == gs,
                                               lane >= EXPERTS_PER_GROUP * j))
        grank = grank + jnp.where(beats, 1.0, 0.0)
    allowed = grank < float(TOPK_GROUPS)

    masked = jnp.where(allowed, scores, NEG_INF)
    sel = jnp.zeros((BLK, NUM_EXPERTS), dtype=jnp.bool_)
    for _ in range(EXPERTS_PER_TOKEN):
        m = jnp.max(masked, axis=1, keepdims=True)
        eq = masked == m
        pfx = jnp.dot(jnp.where(eq, 1.0, 0.0), utri,
                      preferred_element_type=jnp.float32)
        pick = jnp.logical_and(eq, pfx == 0.0)
        sel = jnp.logical_or(sel, pick)
        masked = jnp.where(pick, NEG_INF, masked)

    sel_f = jnp.where(sel, 1.0, 0.0)
    denom = jnp.dot(sel_f * scores, ones64,
                    preferred_element_type=jnp.float32) + 1e-20
    wgt = (scores / denom) * SCALE

    ri = lax.broadcasted_iota(jnp.int32, (BLK, BLK), 0)
    ci = lax.broadcasted_iota(jnp.int32, (BLK, BLK), 1)
    ltri = (ri > ci).astype(jnp.float32)
    excl = jnp.dot(ltri, sel_f, preferred_element_type=jnp.float32)

    @pl.when(blk == 0)
    def _():
        carry_ref[...] = jnp.zeros((8, NUM_EXPERTS), jnp.float32)

    carry = carry_ref[0:1, :]
    rank = excl + carry
    new_carry = carry + jnp.sum(sel_f, axis=0, keepdims=True)
    carry_ref[...] = jnp.broadcast_to(new_carry, (8, NUM_EXPERTS))

    counts_i = jnp.broadcast_to(new_carry, (8, NUM_EXPERTS)).astype(jnp.int32)
    counts_out[...] = counts_i
    lane8 = lax.broadcasted_iota(jnp.int32, (8, NUM_EXPERTS), 1)
    incl = counts_i
    for k in (1, 2, 4, 8, 16, 32):
        incl = incl + jnp.where(lane8 >= k, pltpu.roll(incl, k, axis=1), 0)
    start_out[...] = incl - counts_i

    slot = jnp.dot(sel_f, utri, preferred_element_type=jnp.float32)
    code = lane_f * float(N_TOKENS) + rank
    for s in range(EXPERTS_PER_TOKEN):
        mf = jnp.where(jnp.logical_and(sel, slot == float(s)), 1.0, 0.0)
        code_out[:, s:s + 1] = jnp.sum(mf * code, axis=1,
                                       keepdims=True).astype(jnp.int32)
        wgt_out[:, s:s + 1] = jnp.sum(mf * wgt, axis=1, keepdims=True)


def _routing_call(x, gate_w, bias_2d, interpret=False):
    return pl.pallas_call(
        _routing_body,
        grid=(NBLK,),
        in_specs=[
            pl.BlockSpec((BLK, DIM), lambda i: (i, 0)),
            pl.BlockSpec((DIM, NUM_EXPERTS), lambda i: (0, 0)),
            pl.BlockSpec((1, NUM_EXPERTS), lambda i: (0, 0)),
        ],
        out_specs=[
            pl.BlockSpec((BLK, EXPERTS_PER_TOKEN), lambda i: (i, 0)),
            pl.BlockSpec((BLK, EXPERTS_PER_TOKEN), lambda i: (i, 0)),
            pl.BlockSpec((8, NUM_EXPERTS), lambda i: (0, 0)),
            pl.BlockSpec((8, NUM_EXPERTS), lambda i: (0, 0)),
        ],
        out_shape=[
            jax.ShapeDtypeStruct((N_TOKENS, EXPERTS_PER_TOKEN), jnp.int32),
            jax.ShapeDtypeStruct((N_TOKENS, EXPERTS_PER_TOKEN), jnp.float32),
            jax.ShapeDtypeStruct((8, NUM_EXPERTS), jnp.int32),
            jax.ShapeDtypeStruct((8, NUM_EXPERTS), jnp.int32),
        ],
        scratch_shapes=[pltpu.VMEM((8, NUM_EXPERTS), jnp.float32)],
        compiler_params=pltpu.CompilerParams(
            dimension_semantics=("arbitrary",)),
        interpret=interpret,
    )(x, gate_w, bias_2d)


NSUB = 16
PER_W = TOTAL_SEL // NSUB
CHUNK = 128
NCHUNK = PER_W // CHUNK


@functools.lru_cache(maxsize=1)
def _make_scatter_kernel():
    mesh = plsc.VectorSubcoreMesh(core_axis_name="c", subcore_axis_name="s")

    @functools.partial(
        pl.kernel, mesh=mesh,
        out_type=(jax.ShapeDtypeStruct((TOTAL_SEL,), jnp.float32),
                  jax.ShapeDtypeStruct((TOTAL_SEL,), jnp.int32)),
        scratch_types=[
            pltpu.VMEM((PER_W,), jnp.int32),
            pltpu.VMEM((PER_W,), jnp.float32),
            pltpu.VMEM((PER_W,), jnp.int32),
            pltpu.VMEM((NUM_EXPERTS,), jnp.int32),
            pltpu.VMEM((NCHUNK, CHUNK), jnp.int32),
            pltpu.VMEM_SHARED((TOTAL_SEL,), jnp.float32),
            pltpu.VMEM_SHARED((TOTAL_SEL,), jnp.int32),
            pltpu.SemaphoreType.DMA,
        ],
        compiler_params=pltpu.CompilerParams(needs_layout_passes=False),
    )
    def scatter_kernel(code_hbm, w_hbm, start_hbm, out_w_hbm, out_t_hbm,
                       c_v, wpay_v, tpay_v, st_v, pos_v, shw, sht, sem):
        cid = lax.axis_index("c")
        sid = lax.axis_index("s")
        base = sid * PER_W
        pltpu.sync_copy(code_hbm.at[pl.ds(base, PER_W)], c_v)
        pltpu.sync_copy(start_hbm, st_v)
        iota16 = lax.iota(jnp.int32, 16)
        for v in range(PER_W // 16):
            cv = c_v[pl.ds(v * 16, 16)]
            ev = lax.shift_right_logical(cv, 13)
            rv = jnp.bitwise_and(cv, N_TOKENS - 1)
            pos_v[v // 8, pl.ds((v % 8) * 16, 16)] = rv + plsc.load_gather(
                st_v, [ev])

        @pl.when(cid == 0)
        def _():
            pltpu.sync_copy(w_hbm.at[pl.ds(base, PER_W)], wpay_v)
            copies = []
            for j in range(NCHUNK):
                c = pltpu.make_async_copy(wpay_v.at[pl.ds(j * CHUNK, CHUNK)],
                                          shw.at[pos_v.at[j]], sem)
                c.start()
                copies.append(c)
            for c in copies:
                c.wait()

        @pl.when(cid == 1)
        def _():
            for v in range(PER_W // 16):
                tpay_v[pl.ds(v * 16, 16)] = lax.shift_right_logical(
                    base + v * 16 + iota16, 3)
            copies = []
            for j in range(NCHUNK):
                c = pltpu.make_async_copy(tpay_v.at[pl.ds(j * CHUNK, CHUNK)],
                                          sht.at[pos_v.at[j]], sem)
                c.start()
                copies.append(c)
            for c in copies:
                c.wait()

        plsc.subcore_barrier()

        @pl.when(cid == 0)
        def _():
            pltpu.sync_copy(shw.at[pl.ds(base, PER_W)],
                            out_w_hbm.at[pl.ds(base, PER_W)])

        @pl.when(cid == 1)
        def _():
            pltpu.sync_copy(sht.at[pl.ds(base, PER_W)],
                            out_t_hbm.at[pl.ds(base, PER_W)])

    return scatter_kernel


def kernel(x, gate_w, e_score_correction_bias):
    bias_2d = e_score_correction_bias.reshape(1, NUM_EXPERTS)
    code, wgt, counts, start = _routing_call(x, gate_w, bias_2d)
    out_w, out_t = _make_scatter_kernel()(
        code.reshape(-1), wgt.reshape(-1), start[0])
    return out_w, out_t, counts[0]


# speedup vs baseline: 2.0037x; 2.0037x over previous
import functools

import jax
import jax.numpy as jnp
from jax import lax
from jax.experimental import pallas as pl
from jax.experimental.pallas import tpu as pltpu
from jax.experimental.pallas import tpu_sc as plsc

DIM = 2048
NUM_EXPERTS = 64
EXPERTS_PER_TOKEN = 8
NUM_GROUPS = 8
EXPERTS_PER_GROUP = NUM_EXPERTS // NUM_GROUPS
TOPK_GROUPS = 4
SCALE = 2.5
N_TOKENS = 8192
BLK = 1024
NBLK = N_TOKENS // BLK
TOTAL_SEL = N_TOKENS * EXPERTS_PER_TOKEN

NEG_INF = float("-inf")


def _partner(v, k):
    left = pltpu.roll(v, NUM_EXPERTS - k, axis=1)
    right = pltpu.roll(v, k, axis=1)
    lane = lax.broadcasted_iota(jnp.int32, v.shape, 1)
    return jnp.where(lane & k == 0, left, right)


def _group_reduce(v, op):
    for k in (1, 2, 4):
        v = op(v, _partner(v, k))
    return v


def _select_rows(scores, utri, gtri, gspread):
    m1 = _group_reduce(scores, jnp.maximum)
    eq1 = scores == m1
    pfx1 = jnp.dot(jnp.where(eq1, 1.0, 0.0), gtri,
                   preferred_element_type=jnp.float32)
    foc1 = jnp.logical_and(eq1, pfx1 == 0.0)
    m2 = _group_reduce(jnp.where(foc1, NEG_INF, scores), jnp.maximum)
    gs = m1 + m2

    allowed = jnp.zeros(scores.shape, dtype=jnp.bool_)
    rem = gs
    for _ in range(TOPK_GROUPS):
        m = jnp.max(rem, axis=1, keepdims=True)
        eq = rem == m
        pfx = jnp.dot(jnp.where(eq, 1.0, 0.0), utri,
                      preferred_element_type=jnp.float32)
        pick = jnp.logical_and(eq, pfx == 0.0)
        gsel = jnp.dot(jnp.where(pick, 1.0, 0.0), gspread,
                       preferred_element_type=jnp.float32) > 0.0
        allowed = jnp.logical_or(allowed, gsel)
        rem = jnp.where(gsel, NEG_INF, rem)

    masked = jnp.where(allowed, scores, NEG_INF)
    sel = jnp.zeros(scores.shape, dtype=jnp.bool_)
    denom = jnp.zeros((scores.shape[0], 1), jnp.float32)
    for _ in range(EXPERTS_PER_TOKEN):
        m = jnp.max(masked, axis=1, keepdims=True)
        denom = denom + m
        eq = masked == m
        pfx = jnp.dot(jnp.where(eq, 1.0, 0.0), utri,
                      preferred_element_type=jnp.float32)
        pick = jnp.logical_and(eq, pfx == 0.0)
        sel = jnp.logical_or(sel, pick)
        masked = jnp.where(pick, NEG_INF, masked)

    wgt = (scores / (denom + 1e-20)) * SCALE
    return sel, wgt


NSLAB = 1


def _routing_body(x_ref, w_ref, b_ref,
                  code_out, wgt_out, counts_out, start_out,
                  carry_ref):
    blk = pl.program_id(0)
    logits = jnp.dot(x_ref[...], w_ref[...], preferred_element_type=jnp.float32)
    scores = jax.nn.sigmoid(logits) + b_ref[...]

    lane = lax.broadcasted_iota(jnp.int32, (BLK, NUM_EXPERTS), 1)
    lane_f = lane.astype(jnp.float32)

    ei = lax.broadcasted_iota(jnp.int32, (NUM_EXPERTS, NUM_EXPERTS), 0)
    ej = lax.broadcasted_iota(jnp.int32, (NUM_EXPERTS, NUM_EXPERTS), 1)
    utri = (ei < ej).astype(jnp.float32)
    gtri = ((ei < ej) & (ei // EXPERTS_PER_GROUP == ej // EXPERTS_PER_GROUP)
            ).astype(jnp.float32)
    gspread = (ei // EXPERTS_PER_GROUP == ej // EXPERTS_PER_GROUP
               ).astype(jnp.float32)

    H = BLK // NSLAB
    parts = [_select_rows(scores[i * H:(i + 1) * H], utri, gtri, gspread)
             for i in range(NSLAB)]
    sel = jnp.concatenate([p[0] for p in parts], axis=0)
    wgt = jnp.concatenate([p[1] for p in parts], axis=0)
    sel_f = jnp.where(sel, 1.0, 0.0)

    ri = lax.broadcasted_iota(jnp.int32, (BLK, BLK), 0)
    ci = lax.broadcasted_iota(jnp.int32, (BLK, BLK), 1)
    ltri = (ri > ci).astype(jnp.float32)
    excl = jnp.dot(ltri, sel_f, preferred_element_type=jnp.float32)

    @pl.when(blk == 0)
    def _():
        carry_ref[...] = jnp.zeros((8, NUM_EXPERTS), jnp.float32)

    carry = carry_ref[0:1, :]
    rank = excl + carry
    new_carry = carry + jnp.sum(sel_f, axis=0, keepdims=True)
    carry_ref[...] = jnp.broadcast_to(new_carry, (8, NUM_EXPERTS))

    counts_i = jnp.broadcast_to(new_carry, (8, NUM_EXPERTS)).astype(jnp.int32)
    counts_out[...] = counts_i
    lane8 = lax.broadcasted_iota(jnp.int32, (8, NUM_EXPERTS), 1)
    incl = counts_i
    for k in (1, 2, 4, 8, 16, 32):
        incl = incl + jnp.where(lane8 >= k, pltpu.roll(incl, k, axis=1), 0)
    start_out[...] = incl - counts_i

    slot = jnp.dot(sel_f, utri, preferred_element_type=jnp.float32)
    code = lane_f * float(N_TOKENS) + rank
    code_cols = []
    wgt_cols = []
    for s in range(EXPERTS_PER_TOKEN):
        mf = jnp.where(jnp.logical_and(sel, slot == float(s)), 1.0, 0.0)
        code_cols.append(jnp.sum(mf * code, axis=1, keepdims=True))
        wgt_cols.append(jnp.sum(mf * wgt, axis=1, keepdims=True))
    code_out[...] = jnp.concatenate(code_cols, axis=1).astype(jnp.int32)
    wgt_out[...] = jnp.concatenate(wgt_cols, axis=1)


def _routing_call(x, gate_w, bias_2d, interpret=False):
    return pl.pallas_call(
        _routing_body,
        grid=(NBLK,),
        in_specs=[
            pl.BlockSpec((BLK, DIM), lambda i: (i, 0)),
            pl.BlockSpec((DIM, NUM_EXPERTS), lambda i: (0, 0)),
            pl.BlockSpec((1, NUM_EXPERTS), lambda i: (0, 0)),
        ],
        out_specs=[
            pl.BlockSpec((BLK, EXPERTS_PER_TOKEN), lambda i: (i, 0)),
            pl.BlockSpec((BLK, EXPERTS_PER_TOKEN), lambda i: (i, 0)),
            pl.BlockSpec((8, NUM_EXPERTS), lambda i: (0, 0)),
            pl.BlockSpec((8, NUM_EXPERTS), lambda i: (0, 0)),
        ],
        out_shape=[
            jax.ShapeDtypeStruct((N_TOKENS, EXPERTS_PER_TOKEN), jnp.int32),
            jax.ShapeDtypeStruct((N_TOKENS, EXPERTS_PER_TOKEN), jnp.float32),
            jax.ShapeDtypeStruct((8, NUM_EXPERTS), jnp.int32),
            jax.ShapeDtypeStruct((8, NUM_EXPERTS), jnp.int32),
        ],
        scratch_shapes=[pltpu.VMEM((8, NUM_EXPERTS), jnp.float32)],
        compiler_params=pltpu.CompilerParams(
            dimension_semantics=("arbitrary",)),
        interpret=interpret,
    )(x, gate_w, bias_2d)


NSUB = 16
PER_W = TOTAL_SEL // NSUB
CHUNK = 128
NCHUNK = PER_W // CHUNK


@functools.lru_cache(maxsize=1)
def _make_scatter_kernel():
    mesh = plsc.VectorSubcoreMesh(core_axis_name="c", subcore_axis_name="s")

    @functools.partial(
        pl.kernel, mesh=mesh,
        out_type=(jax.ShapeDtypeStruct((TOTAL_SEL,), jnp.float32),
                  jax.ShapeDtypeStruct((TOTAL_SEL,), jnp.int32)),
        scratch_types=[
            pltpu.VMEM((PER_W,), jnp.int32),
            pltpu.VMEM((PER_W,), jnp.float32),
            pltpu.VMEM((PER_W,), jnp.int32),
            pltpu.VMEM((NUM_EXPERTS,), jnp.int32),
            pltpu.VMEM((NCHUNK, CHUNK), jnp.int32),
            pltpu.VMEM_SHARED((TOTAL_SEL,), jnp.float32),
            pltpu.VMEM_SHARED((TOTAL_SEL,), jnp.int32),
            pltpu.SemaphoreType.DMA,
        ],
        compiler_params=pltpu.CompilerParams(needs_layout_passes=False),
    )
    def scatter_kernel(code_hbm, w_hbm, start_hbm, out_w_hbm, out_t_hbm,
                       c_v, wpay_v, tpay_v, st_v, pos_v, shw, sht, sem):
        cid = lax.axis_index("c")
        sid = lax.axis_index("s")
        base = sid * PER_W
        pltpu.sync_copy(code_hbm.at[pl.ds(base, PER_W)], c_v)
        pltpu.sync_copy(start_hbm, st_v)
        iota16 = lax.iota(jnp.int32, 16)
        for v in range(PER_W // 16):
            cv = c_v[pl.ds(v * 16, 16)]
            ev = lax.shift_right_logical(cv, 13)
            rv = jnp.bitwise_and(cv, N_TOKENS - 1)
            pos_v[v // 8, pl.ds((v % 8) * 16, 16)] = rv + plsc.load_gather(
                st_v, [ev])

        @pl.when(cid == 0)
        def _():
            pltpu.sync_copy(w_hbm.at[pl.ds(base, PER_W)], wpay_v)
            copies = []
            for j in range(NCHUNK):
                c = pltpu.make_async_copy(wpay_v.at[pl.ds(j * CHUNK, CHUNK)],
                                          shw.at[pos_v.at[j]], sem)
                c.start()
                copies.append(c)
            for c in copies:
                c.wait()

        @pl.when(cid == 1)
        def _():
            for v in range(PER_W // 16):
                tpay_v[pl.ds(v * 16, 16)] = lax.shift_right_logical(
                    base + v * 16 + iota16, 3)
            copies = []
            for j in range(NCHUNK):
                c = pltpu.make_async_copy(tpay_v.at[pl.ds(j * CHUNK, CHUNK)],
                                          sht.at[pos_v.at[j]], sem)
                c.start()
                copies.append(c)
            for c in copies:
                c.wait()

        plsc.subcore_barrier()

        @pl.when(cid == 0)
        def _():
            pltpu.sync_copy(shw.at[pl.ds(base, PER_W)],
                            out_w_hbm.at[pl.ds(base, PER_W)])

        @pl.when(cid == 1)
        def _():
            pltpu.sync_copy(sht.at[pl.ds(base, PER_W)],
                            out_t_hbm.at[pl.ds(base, PER_W)])

    return scatter_kernel


def kernel(x, gate_w, e_score_correction_bias):
    bias_2d = e_score_correction_bias.reshape(1, NUM_EXPERTS)
    code, wgt, counts, start = _routing_call(x, gate_w, bias_2d)
    out_w, out_t = _make_scatter_kernel()(
        code.reshape(-1), wgt.reshape(-1), start[0])
    return out_w, out_t, counts[0]


# speedup vs baseline: 2.1701x; 1.0831x over previous
import functools

import jax
import jax.numpy as jnp
from jax import lax
from jax.experimental import pallas as pl
from jax.experimental.pallas import tpu as pltpu
from jax.experimental.pallas import tpu_sc as plsc

DIM = 2048
NUM_EXPERTS = 64
EXPERTS_PER_TOKEN = 8
NUM_GROUPS = 8
EXPERTS_PER_GROUP = NUM_EXPERTS // NUM_GROUPS
TOPK_GROUPS = 4
SCALE = 2.5
N_TOKENS = 8192
BLK = 1024
NBLK = N_TOKENS // BLK
TOTAL_SEL = N_TOKENS * EXPERTS_PER_TOKEN

NEG_INF = float("-inf")


def _partner(v, k):
    left = pltpu.roll(v, NUM_EXPERTS - k, axis=1)
    right = pltpu.roll(v, k, axis=1)
    lane = lax.broadcasted_iota(jnp.int32, v.shape, 1)
    return jnp.where(lane & k == 0, left, right)


def _group_reduce(v, op):
    for k in (1, 2, 4):
        v = op(v, _partner(v, k))
    return v


def _select_rows(scores, utri, gspread):
    m1 = scores
    m2 = jnp.full(scores.shape, NEG_INF, jnp.float32)
    for k in (1, 2, 4):
        p1 = _partner(m1, k)
        p2 = _partner(m2, k)
        m1, m2 = (jnp.maximum(m1, p1),
                  jnp.maximum(jnp.minimum(m1, p1), jnp.maximum(m2, p2)))
    gs = m1 + m2

    allowed = jnp.zeros(scores.shape, dtype=jnp.bool_)
    rem = gs
    for _ in range(TOPK_GROUPS):
        m = jnp.max(rem, axis=1, keepdims=True)
        eq = rem == m
        pfx = jnp.dot(jnp.where(eq, 1.0, 0.0), utri,
                      preferred_element_type=jnp.float32)
        pick = jnp.logical_and(eq, pfx == 0.0)
        gsel = jnp.dot(jnp.where(pick, 1.0, 0.0), gspread,
                       preferred_element_type=jnp.float32) > 0.0
        allowed = jnp.logical_or(allowed, gsel)
        rem = jnp.where(gsel, NEG_INF, rem)

    masked = jnp.where(allowed, scores, NEG_INF)
    sel = jnp.zeros(scores.shape, dtype=jnp.bool_)
    denom = jnp.zeros((scores.shape[0], 1), jnp.float32)
    picks = []
    maxes = []
    for _ in range(EXPERTS_PER_TOKEN):
        m = jnp.max(masked, axis=1, keepdims=True)
        denom = denom + m
        eq = masked == m
        pfx = jnp.dot(jnp.where(eq, 1.0, 0.0), utri,
                      preferred_element_type=jnp.float32)
        pick = jnp.logical_and(eq, pfx == 0.0)
        picks.append(pick)
        maxes.append(m)
        sel = jnp.logical_or(sel, pick)
        masked = jnp.where(pick, NEG_INF, masked)

    return picks, maxes, denom, sel


NSLAB = 1


def _routing_body(x_ref, w_ref, b_ref,
                  code_out, wgt_out, counts_out, start_out,
                  carry_ref):
    blk = pl.program_id(0)
    logits = jnp.dot(x_ref[...], w_ref[...], preferred_element_type=jnp.float32)
    scores = jax.nn.sigmoid(logits) + b_ref[...]

    lane = lax.broadcasted_iota(jnp.int32, (BLK, NUM_EXPERTS), 1)
    lane_f = lane.astype(jnp.float32)

    ei = lax.broadcasted_iota(jnp.int32, (NUM_EXPERTS, NUM_EXPERTS), 0)
    ej = lax.broadcasted_iota(jnp.int32, (NUM_EXPERTS, NUM_EXPERTS), 1)
    utri = (ei < ej).astype(jnp.float32)
    gspread = (ei // EXPERTS_PER_GROUP == ej // EXPERTS_PER_GROUP
               ).astype(jnp.float32)

    picks, maxes, denom, sel = _select_rows(scores, utri, gspread)
    sel_f = jnp.where(sel, 1.0, 0.0)

    ri = lax.broadcasted_iota(jnp.int32, (BLK, BLK), 0)
    ci = lax.broadcasted_iota(jnp.int32, (BLK, BLK), 1)
    ltri = (ri > ci).astype(jnp.float32)
    excl = jnp.dot(ltri, sel_f, preferred_element_type=jnp.float32)

    @pl.when(blk == 0)
    def _():
        carry_ref[...] = jnp.zeros((8, NUM_EXPERTS), jnp.float32)

    carry = carry_ref[0:1, :]
    rank = excl + carry
    new_carry = carry + jnp.sum(sel_f, axis=0, keepdims=True)
    carry_ref[...] = jnp.broadcast_to(new_carry, (8, NUM_EXPERTS))

    counts_i = jnp.broadcast_to(new_carry, (8, NUM_EXPERTS)).astype(jnp.int32)
    counts_out[...] = counts_i
    lane8 = lax.broadcasted_iota(jnp.int32, (8, NUM_EXPERTS), 1)
    incl = counts_i
    for k in (1, 2, 4, 8, 16, 32):
        incl = incl + jnp.where(lane8 >= k, pltpu.roll(incl, k, axis=1), 0)
    start_out[...] = incl - counts_i

    code = lane_f * float(N_TOKENS) + rank
    code_cols = [jnp.sum(jnp.where(p, code, 0.0), axis=1, keepdims=True)
                 for p in picks]
    code_out[...] = jnp.concatenate(code_cols, axis=1).astype(jnp.int32)
    wgt_out[...] = (jnp.concatenate(maxes, axis=1)
                    / (denom + 1e-20)) * SCALE


def _routing_call(x, gate_w, bias_2d, interpret=False):
    return pl.pallas_call(
        _routing_body,
        grid=(NBLK,),
        in_specs=[
            pl.BlockSpec((BLK, DIM), lambda i: (i, 0)),
            pl.BlockSpec((DIM, NUM_EXPERTS), lambda i: (0, 0)),
            pl.BlockSpec((1, NUM_EXPERTS), lambda i: (0, 0)),
        ],
        out_specs=[
            pl.BlockSpec((BLK, EXPERTS_PER_TOKEN), lambda i: (i, 0)),
            pl.BlockSpec((BLK, EXPERTS_PER_TOKEN), lambda i: (i, 0)),
            pl.BlockSpec((8, NUM_EXPERTS), lambda i: (0, 0)),
            pl.BlockSpec((8, NUM_EXPERTS), lambda i: (0, 0)),
        ],
        out_shape=[
            jax.ShapeDtypeStruct((N_TOKENS, EXPERTS_PER_TOKEN), jnp.int32),
            jax.ShapeDtypeStruct((N_TOKENS, EXPERTS_PER_TOKEN), jnp.float32),
            jax.ShapeDtypeStruct((8, NUM_EXPERTS), jnp.int32),
            jax.ShapeDtypeStruct((8, NUM_EXPERTS), jnp.int32),
        ],
        scratch_shapes=[pltpu.VMEM((8, NUM_EXPERTS), jnp.float32)],
        compiler_params=pltpu.CompilerParams(
            dimension_semantics=("arbitrary",)),
        interpret=interpret,
    )(x, gate_w, bias_2d)


NSUB = 16
PER_W = TOTAL_SEL // NSUB
CHUNK = 128
NCHUNK = PER_W // CHUNK


@functools.lru_cache(maxsize=1)
def _make_scatter_kernel():
    mesh = plsc.VectorSubcoreMesh(core_axis_name="c", subcore_axis_name="s")

    @functools.partial(
        pl.kernel, mesh=mesh,
        out_type=(jax.ShapeDtypeStruct((TOTAL_SEL,), jnp.float32),
                  jax.ShapeDtypeStruct((TOTAL_SEL,), jnp.int32)),
        scratch_types=[
            pltpu.VMEM((PER_W,), jnp.int32),
            pltpu.VMEM((PER_W,), jnp.float32),
            pltpu.VMEM((PER_W,), jnp.int32),
            pltpu.VMEM((NUM_EXPERTS,), jnp.int32),
            pltpu.VMEM((NCHUNK, CHUNK), jnp.int32),
            pltpu.VMEM_SHARED((TOTAL_SEL,), jnp.float32),
            pltpu.VMEM_SHARED((TOTAL_SEL,), jnp.int32),
            pltpu.SemaphoreType.DMA,
        ],
        compiler_params=pltpu.CompilerParams(needs_layout_passes=False),
    )
    def scatter_kernel(code_hbm, w_hbm, start_hbm, out_w_hbm, out_t_hbm,
                       c_v, wpay_v, tpay_v, st_v, pos_v, shw, sht, sem):
        cid = lax.axis_index("c")
        sid = lax.axis_index("s")
        base = sid * PER_W
        pltpu.sync_copy(code_hbm.at[pl.ds(base, PER_W)], c_v)
        pltpu.sync_copy(start_hbm, st_v)
        iota16 = lax.iota(jnp.int32, 16)
        for v in range(PER_W // 16):
            cv = c_v[pl.ds(v * 16, 16)]
            ev = lax.shift_right_logical(cv, 13)
            rv = jnp.bitwise_and(cv, N_TOKENS - 1)
            pos_v[v // 8, pl.ds((v % 8) * 16, 16)] = rv + plsc.load_gather(
                st_v, [ev])

        @pl.when(cid == 0)
        def _():
            pltpu.sync_copy(w_hbm.at[pl.ds(base, PER_W)], wpay_v)
            copies = []
            for j in range(NCHUNK):
                c = pltpu.make_async_copy(wpay_v.at[pl.ds(j * CHUNK, CHUNK)],
                                          shw.at[pos_v.at[j]], sem)
                c.start()
                copies.append(c)
            for c in copies:
                c.wait()

        @pl.when(cid == 1)
        def _():
            for v in range(PER_W // 16):
                tpay_v[pl.ds(v * 16, 16)] = lax.shift_right_logical(
                    base + v * 16 + iota16, 3)
            copies = []
            for j in range(NCHUNK):
                c = pltpu.make_async_copy(tpay_v.at[pl.ds(j * CHUNK, CHUNK)],
                                          sht.at[pos_v.at[j]], sem)
                c.start()
                copies.append(c)
            for c in copies:
                c.wait()

        plsc.subcore_barrier()

        @pl.when(cid == 0)
        def _():
            pltpu.sync_copy(shw.at[pl.ds(base, PER_W)],
                            out_w_hbm.at[pl.ds(base, PER_W)])

        @pl.when(cid == 1)
        def _():
            pltpu.sync_copy(sht.at[pl.ds(base, PER_W)],
                            out_t_hbm.at[pl.ds(base, PER_W)])

    return scatter_kernel


def kernel(x, gate_w, e_score_correction_bias):
    bias_2d = e_score_correction_bias.reshape(1, NUM_EXPERTS)
    code, wgt, counts, start = _routing_call(x, gate_w, bias_2d)
    out_w, out_t = _make_scatter_kernel()(
        code.reshape(-1), wgt.reshape(-1), start[0])
    return out_w, out_t, counts[0]


# speedup vs baseline: 2.2933x; 1.0568x over previous
import functools

import jax
import jax.numpy as jnp
from jax import lax
from jax.experimental import pallas as pl
from jax.experimental.pallas import tpu as pltpu
from jax.experimental.pallas import tpu_sc as plsc

DIM = 2048
NUM_EXPERTS = 64
EXPERTS_PER_TOKEN = 8
NUM_GROUPS = 8
EXPERTS_PER_GROUP = NUM_EXPERTS // NUM_GROUPS
TOPK_GROUPS = 4
SCALE = 2.5
N_TOKENS = 8192
BLK = 1024
NBLK = N_TOKENS // BLK
TOTAL_SEL = N_TOKENS * EXPERTS_PER_TOKEN

NEG_INF = float("-inf")


def _select_rows(scores, utri, gspread):
    m1 = scores
    m2 = jnp.full(scores.shape, NEG_INF, jnp.float32)
    for k in (1, 2, 4):
        p1 = pltpu.roll(m1, NUM_EXPERTS - k, axis=1)
        p2 = pltpu.roll(m2, NUM_EXPERTS - k, axis=1)
        m1, m2 = (jnp.maximum(m1, p1),
                  jnp.maximum(jnp.minimum(m1, p1), jnp.maximum(m2, p2)))
    lane = lax.broadcasted_iota(jnp.int32, scores.shape, 1)
    gs = jnp.where(lane % EXPERTS_PER_GROUP == 0, m1 + m2, NEG_INF)

    gpicks = jnp.zeros(scores.shape, jnp.float32)
    rem = gs
    for _ in range(TOPK_GROUPS):
        m = jnp.max(rem, axis=1, keepdims=True)
        eq = rem == m
        pfx = jnp.dot(jnp.where(eq, 1.0, 0.0), utri,
                      preferred_element_type=jnp.float32)
        pick = jnp.logical_and(eq, pfx == 0.0)
        gpicks = gpicks + jnp.where(pick, 1.0, 0.0)
        rem = jnp.where(pick, NEG_INF, rem)
    allowed = jnp.dot(gpicks, gspread,
                      preferred_element_type=jnp.float32) > 0.0

    masked = jnp.where(allowed, scores, NEG_INF)
    sel = jnp.zeros(scores.shape, dtype=jnp.bool_)
    denom = jnp.zeros((scores.shape[0], 1), jnp.float32)
    picks = []
    maxes = []
    for _ in range(EXPERTS_PER_TOKEN):
        m = jnp.max(masked, axis=1, keepdims=True)
        denom = denom + m
        eq = masked == m
        pfx = jnp.dot(jnp.where(eq, 1.0, 0.0), utri,
                      preferred_element_type=jnp.float32)
        pick = jnp.logical_and(eq, pfx == 0.0)
        picks.append(pick)
        maxes.append(m)
        sel = jnp.logical_or(sel, pick)
        masked = jnp.where(pick, NEG_INF, masked)

    return picks, maxes, denom, sel


NSLAB = 1


def _routing_body(x_ref, w_ref, b_ref,
                  code_out, wgt_out, counts_out, start_out,
                  carry_ref):
    blk = pl.program_id(0)
    logits = jnp.dot(x_ref[...], w_ref[...], preferred_element_type=jnp.float32)
    scores = jax.nn.sigmoid(logits) + b_ref[...]

    lane = lax.broadcasted_iota(jnp.int32, (BLK, NUM_EXPERTS), 1)
    lane_f = lane.astype(jnp.float32)

    ei = lax.broadcasted_iota(jnp.int32, (NUM_EXPERTS, NUM_EXPERTS), 0)
    ej = lax.broadcasted_iota(jnp.int32, (NUM_EXPERTS, NUM_EXPERTS), 1)
    utri = (ei < ej).astype(jnp.float32)
    gspread = (ei // EXPERTS_PER_GROUP == ej // EXPERTS_PER_GROUP
               ).astype(jnp.float32)

    picks, maxes, denom, sel = _select_rows(scores, utri, gspread)
    sel_f = jnp.where(sel, 1.0, 0.0)

    ri = lax.broadcasted_iota(jnp.int32, (BLK, BLK), 0)
    ci = lax.broadcasted_iota(jnp.int32, (BLK, BLK), 1)
    ltri = (ri > ci).astype(jnp.float32)
    excl = jnp.dot(ltri, sel_f, preferred_element_type=jnp.float32)

    @pl.when(blk == 0)
    def _():
        carry_ref[...] = jnp.zeros((8, NUM_EXPERTS), jnp.float32)

    carry = carry_ref[0:1, :]
    rank = excl + carry
    new_carry = carry + jnp.sum(sel_f, axis=0, keepdims=True)
    carry_ref[...] = jnp.broadcast_to(new_carry, (8, NUM_EXPERTS))

    counts_i = jnp.broadcast_to(new_carry, (8, NUM_EXPERTS)).astype(jnp.int32)
    counts_out[...] = counts_i
    lane8 = lax.broadcasted_iota(jnp.int32, (8, NUM_EXPERTS), 1)
    incl = counts_i
    for k in (1, 2, 4, 8, 16, 32):
        incl = incl + jnp.where(lane8 >= k, pltpu.roll(incl, k, axis=1), 0)
    start_out[...] = incl - counts_i

    code = lane_f * float(N_TOKENS) + rank
    code_cols = [jnp.sum(jnp.where(p, code, 0.0), axis=1, keepdims=True)
                 for p in picks]
    code_out[...] = jnp.concatenate(code_cols, axis=1).astype(jnp.int32)
    wgt_out[...] = (jnp.concatenate(maxes, axis=1)
                    / (denom + 1e-20)) * SCALE


def _routing_call(x, gate_w, bias_2d, interpret=False):
    return pl.pallas_call(
        _routing_body,
        grid=(NBLK,),
        in_specs=[
            pl.BlockSpec((BLK, DIM), lambda i: (i, 0)),
            pl.BlockSpec((DIM, NUM_EXPERTS), lambda i: (0, 0)),
            pl.BlockSpec((1, NUM_EXPERTS), lambda i: (0, 0)),
        ],
        out_specs=[
            pl.BlockSpec((BLK, EXPERTS_PER_TOKEN), lambda i: (i, 0)),
            pl.BlockSpec((BLK, EXPERTS_PER_TOKEN), lambda i: (i, 0)),
            pl.BlockSpec((8, NUM_EXPERTS), lambda i: (0, 0)),
            pl.BlockSpec((8, NUM_EXPERTS), lambda i: (0, 0)),
        ],
        out_shape=[
            jax.ShapeDtypeStruct((N_TOKENS, EXPERTS_PER_TOKEN), jnp.int32),
            jax.ShapeDtypeStruct((N_TOKENS, EXPERTS_PER_TOKEN), jnp.float32),
            jax.ShapeDtypeStruct((8, NUM_EXPERTS), jnp.int32),
            jax.ShapeDtypeStruct((8, NUM_EXPERTS), jnp.int32),
        ],
        scratch_shapes=[pltpu.VMEM((8, NUM_EXPERTS), jnp.float32)],
        compiler_params=pltpu.CompilerParams(
            dimension_semantics=("arbitrary",)),
        interpret=interpret,
    )(x, gate_w, bias_2d)


NSUB = 16
PER_W = TOTAL_SEL // NSUB
CHUNK = 128
NCHUNK = PER_W // CHUNK


@functools.lru_cache(maxsize=1)
def _make_scatter_kernel():
    mesh = plsc.VectorSubcoreMesh(core_axis_name="c", subcore_axis_name="s")

    @functools.partial(
        pl.kernel, mesh=mesh,
        out_type=(jax.ShapeDtypeStruct((TOTAL_SEL,), jnp.float32),
                  jax.ShapeDtypeStruct((TOTAL_SEL,), jnp.int32)),
        scratch_types=[
            pltpu.VMEM((PER_W,), jnp.int32),
            pltpu.VMEM((PER_W,), jnp.float32),
            pltpu.VMEM((PER_W,), jnp.int32),
            pltpu.VMEM((NUM_EXPERTS,), jnp.int32),
            pltpu.VMEM((NCHUNK, CHUNK), jnp.int32),
            pltpu.VMEM_SHARED((TOTAL_SEL,), jnp.float32),
            pltpu.VMEM_SHARED((TOTAL_SEL,), jnp.int32),
            pltpu.SemaphoreType.DMA,
        ],
        compiler_params=pltpu.CompilerParams(needs_layout_passes=False),
    )
    def scatter_kernel(code_hbm, w_hbm, start_hbm, out_w_hbm, out_t_hbm,
                       c_v, wpay_v, tpay_v, st_v, pos_v, shw, sht, sem):
        cid = lax.axis_index("c")
        sid = lax.axis_index("s")
        base = sid * PER_W
        pltpu.sync_copy(code_hbm.at[pl.ds(base, PER_W)], c_v)
        pltpu.sync_copy(start_hbm, st_v)
        iota16 = lax.iota(jnp.int32, 16)
        for v in range(PER_W // 16):
            cv = c_v[pl.ds(v * 16, 16)]
            ev = lax.shift_right_logical(cv, 13)
            rv = jnp.bitwise_and(cv, N_TOKENS - 1)
            pos_v[v // 8, pl.ds((v % 8) * 16, 16)] = rv + plsc.load_gather(
                st_v, [ev])

        @pl.when(cid == 0)
        def _():
            pltpu.sync_copy(w_hbm.at[pl.ds(base, PER_W)], wpay_v)
            copies = []
            for j in range(NCHUNK):
                c = pltpu.make_async_copy(wpay_v.at[pl.ds(j * CHUNK, CHUNK)],
                                          shw.at[pos_v.at[j]], sem)
                c.start()
                copies.append(c)
            for c in copies:
                c.wait()

        @pl.when(cid == 1)
        def _():
            for v in range(PER_W // 16):
                tpay_v[pl.ds(v * 16, 16)] = lax.shift_right_logical(
                    base + v * 16 + iota16, 3)
            copies = []
            for j in range(NCHUNK):
                c = pltpu.make_async_copy(tpay_v.at[pl.ds(j * CHUNK, CHUNK)],
                                          sht.at[pos_v.at[j]], sem)
                c.start()
                copies.append(c)
            for c in copies:
                c.wait()

        plsc.subcore_barrier()

        @pl.when(cid == 0)
        def _():
            pltpu.sync_copy(shw.at[pl.ds(base, PER_W)],
                            out_w_hbm.at[pl.ds(base, PER_W)])

        @pl.when(cid == 1)
        def _():
            pltpu.sync_copy(sht.at[pl.ds(base, PER_W)],
                            out_t_hbm.at[pl.ds(base, PER_W)])

    return scatter_kernel


def kernel(x, gate_w, e_score_correction_bias):
    bias_2d = e_score_correction_bias.reshape(1, NUM_EXPERTS)
    code, wgt, counts, start = _routing_call(x, gate_w, bias_2d)
    out_w, out_t = _make_scatter_kernel()(
        code.reshape(-1), wgt.reshape(-1), start[0])
    return out_w, out_t, counts[0]
